# baseline (speedup 1.0000x reference)
import functools

import jax
import jax.numpy as jnp
from jax import lax
from jax.experimental import pallas as pl
from jax.experimental.pallas import tpu as pltpu
from jax.experimental.pallas import tpu_sc as plsc

B, S, D = 1024, 200, 128
PAD = 21
VOCAB, MAXPOS = 32, 256
EPS = 1e-12
NC, NS = 2, 16
NW = NC * NS
RW = B // NW
SPAD = 208
NCHUNK = SPAD // 16
NK = D // 16


def _perm(x, idx):
    return x.at[idx].get(mode="promise_in_bounds")


def _splat_sum(x):
    lane = lax.iota(jnp.int32, 16)
    for k in range(4):
        x = x + _perm(x, lane ^ (1 << k))
    return x


def _prefix_sum(x):
    lane = lax.iota(jnp.int32, 16)
    for k in range(4):
        sh = _perm(x, jnp.maximum(lane - (1 << k), 0))
        x = x + jnp.where(lane >= (1 << k), sh, jnp.int32(0))
    return x


def _rsqrt(x):
    i = lax.bitcast_convert_type(x, jnp.int32)
    i = jnp.int32(0x5F3759DF) - lax.shift_right_logical(i, 1)
    y = lax.bitcast_convert_type(i, jnp.float32)
    for _ in range(3):
        y = y * (1.5 - 0.5 * x * y * y)
    return y


def _body(src_h, aa_h, pos_h, ga_h, be_h, out_h,
          aa_v, pos_v, ga_v, be_v, src_v, obuf, sem_in, sem_out):
    wid = lax.axis_index("s") * NC + lax.axis_index("c")
    base_row = wid * RW

    stage = [
        pltpu.make_async_copy(aa_h, aa_v, sem_in),
        pltpu.make_async_copy(pos_h, pos_v, sem_in),
        pltpu.make_async_copy(ga_h, ga_v, sem_in),
        pltpu.make_async_copy(be_h, be_v, sem_in),
        pltpu.make_async_copy(src_h.at[pl.ds(base_row * S, RW * S)],
                              src_v.at[pl.ds(0, RW * S)], sem_in),
    ]
    for c in stage:
        c.start()
    for c in stage:
        c.wait()

    gvec = [ga_v[pl.ds(k * 16, 16)] for k in range(NK)]
    bvec = [be_v[pl.ds(k * 16, 16)] for k in range(NK)]
    last = jnp.full((16,), 15, jnp.int32)

    def row_body(r, _):
        bidx = lax.rem(r, 2)

        @pl.when(r >= 2)
        def _wait_prev():
            pltpu.make_async_copy(obuf.at[bidx, pl.ds(0, S)],
                                  out_h.at[base_row + r - 2], sem_out).wait()

        def chunk_body(i, carry):
            tb = i * 16
            sv_raw = src_v[pl.ds(r * S + tb, 16)]
            sv = jnp.minimum(jnp.maximum(sv_raw, jnp.int32(0)),
                             jnp.int32(VOCAB - 1))
            m = jnp.where(sv_raw == PAD, jnp.int32(0), jnp.int32(1))
            c = _prefix_sum(m) + carry
            pv = jnp.where(m == 1, c, jnp.int32(0))
            pv = jnp.minimum(pv, jnp.int32(MAXPOS - 1))
            for l in range(16):
                s = sv[l]
                p = pv[l]
                e = [aa_v[s, pl.ds(k * 16, 16)] + pos_v[p, pl.ds(k * 16, 16)]
                     for k in range(NK)]
                tot = ((e[0] + e[1]) + (e[2] + e[3])) + \
                      ((e[4] + e[5]) + (e[6] + e[7]))
                sq = [ek * ek for ek in e]
                sqt = ((sq[0] + sq[1]) + (sq[2] + sq[3])) + \
                      ((sq[4] + sq[5]) + (sq[6] + sq[7]))
                mean = _splat_sum(tot) * (1.0 / D)
                ex2 = _splat_sum(sqt) * (1.0 / D)
                var = jnp.maximum(ex2 - mean * mean, 0.0) + EPS
                rstd = _rsqrt(var)
                for k in range(NK):
                    obuf[bidx, tb + l, pl.ds(k * 16, 16)] = \
                        (e[k] - mean) * rstd * gvec[k] + bvec[k]
            return _perm(c, last)

        lax.fori_loop(0, NCHUNK, chunk_body, jnp.zeros((16,), jnp.int32), unroll=2)
        pltpu.make_async_copy(obuf.at[bidx, pl.ds(0, S)],
                              out_h.at[base_row + r], sem_out).start()
        return 0

    lax.fori_loop(0, RW, row_body, 0)
    pltpu.make_async_copy(obuf.at[0, pl.ds(0, S)],
                          out_h.at[base_row + RW - 2], sem_out).wait()
    pltpu.make_async_copy(obuf.at[1, pl.ds(0, S)],
                          out_h.at[base_row + RW - 1], sem_out).wait()


_emb = functools.partial(
    pl.kernel,
    out_type=jax.ShapeDtypeStruct((B, S, D), jnp.float32),
    mesh=plsc.VectorSubcoreMesh(core_axis_name="c", subcore_axis_name="s"),
    scratch_types=[
        pltpu.VMEM((VOCAB, D), jnp.float32),
        pltpu.VMEM((MAXPOS, D), jnp.float32),
        pltpu.VMEM((D,), jnp.float32),
        pltpu.VMEM((D,), jnp.float32),
        pltpu.VMEM((RW * S + 64,), jnp.int32),
        pltpu.VMEM((2, SPAD, D), jnp.float32),
        pltpu.SemaphoreType.DMA,
        pltpu.SemaphoreType.DMA,
    ],
)(_body)


def kernel(src, AA_emb, Pos_emb, ln_gamma, ln_beta):
    src_flat = src.reshape(-1).astype(jnp.int32)
    return _emb(src_flat, AA_emb, Pos_emb, ln_gamma, ln_beta)


# speedup vs baseline: 1.6761x; 1.6761x over previous
import functools

import jax
import jax.numpy as jnp
from jax import lax
from jax.experimental import pallas as pl
from jax.experimental.pallas import tpu as pltpu
from jax.experimental.pallas import tpu_sc as plsc

B, S, D = 1024, 200, 128
PAD = 21
VOCAB, MAXPOS = 32, 256
EPS = 1e-12
NC, NS = 2, 16
NW = NC * NS
RW = B // NW
SPAD = 208
NCHUNK = SPAD // 16
NK = D // 16
H0, H1 = 112, SPAD - 112


def _perm(x, idx):
    return x.at[idx].get(mode="promise_in_bounds")


def _prefix_sum(x):
    lane = lax.iota(jnp.int32, 16)
    for k in range(4):
        sh = _perm(x, jnp.maximum(lane - (1 << k), 0))
        x = x + jnp.where(lane >= (1 << k), sh, jnp.int32(0))
    return x


def _body(src_h, aa_h, pos_h, ga_h, be_h, m_h, r_h, out_h,
          aa_v, pos_v, ga_v, be_v, src_v, pidx_v, mbuf, rbuf, obuf,
          sem_in, sem_mr, sem_out):
    wid = lax.axis_index("s") * NC + lax.axis_index("c")
    base_row = wid * RW

    stage = [
        pltpu.make_async_copy(aa_h, aa_v, sem_in),
        pltpu.make_async_copy(pos_h, pos_v, sem_in),
        pltpu.make_async_copy(ga_h, ga_v, sem_in),
        pltpu.make_async_copy(be_h, be_v, sem_in),
        pltpu.make_async_copy(src_h.at[pl.ds(base_row * S, RW * S)],
                              src_v.at[pl.ds(0, RW * S)], sem_in),
    ]
    for c in stage:
        c.start()
    for c in stage:
        c.wait()

    gvec = [ga_v[pl.ds(k * 16, 16)] for k in range(NK)]
    bvec = [be_v[pl.ds(k * 16, 16)] for k in range(NK)]
    last = jnp.full((16,), 15, jnp.int32)

    def compute_pair_idx(row):
        b = lax.rem(row, 2)

        def och(i, carry):
            tb = i * 16
            sv_raw = src_v[pl.ds(row * S + tb, 16)]
            sv = jnp.minimum(jnp.maximum(sv_raw, jnp.int32(0)),
                             jnp.int32(VOCAB - 1))
            m = jnp.where(sv_raw == PAD, jnp.int32(0), jnp.int32(1))
            c = _prefix_sum(m) + carry
            pv = jnp.where(m == 1, c, jnp.int32(0))
            pv = jnp.minimum(pv, jnp.int32(MAXPOS - 1))
            pidx_v[pl.ds(b * SPAD + tb, 16)] = sv * jnp.int32(MAXPOS) + pv
            return _perm(c, last)

        lax.fori_loop(0, NCHUNK, och, jnp.zeros((16,), jnp.int32))

    def mr_copies(row):
        b = lax.rem(row, 2)
        cps = []
        for off, ln in ((0, H0), (H0, H1)):
            cps.append(pltpu.make_async_copy(
                m_h.at[pidx_v.at[pl.ds(b * SPAD + off, ln)]],
                mbuf.at[pl.ds(b * SPAD + off, ln)], sem_mr))
            cps.append(pltpu.make_async_copy(
                r_h.at[pidx_v.at[pl.ds(b * SPAD + off, ln)]],
                rbuf.at[pl.ds(b * SPAD + off, ln)], sem_mr))
        return cps

    compute_pair_idx(0)
    for c in mr_copies(0):
        c.start()

    def row_body(r, _):
        bidx = lax.rem(r, 2)

        for c in mr_copies(r):
            c.wait()

        @pl.when(r >= 2)
        def _wait_prev():
            pltpu.make_async_copy(obuf.at[bidx, pl.ds(0, S)],
                                  out_h.at[base_row + r - 2], sem_out).wait()

        @pl.when(r + 1 < RW)
        def _prefetch_next():
            compute_pair_idx(r + 1)
            for c in mr_copies(r + 1):
                c.start()

        def chunk_body(i, _2):
            tb = i * 16
            pc = pidx_v[pl.ds(bidx * SPAD + tb, 16)]
            sv = lax.shift_right_logical(pc, 8)
            pv = jnp.bitwise_and(pc, jnp.int32(MAXPOS - 1))
            mv = mbuf[pl.ds(bidx * SPAD + tb, 16)]
            rv = rbuf[pl.ds(bidx * SPAD + tb, 16)]
            for l in range(16):
                s = sv[l]
                p = pv[l]
                ml = mv[l]
                rl = rv[l]
                for k in range(NK):
                    e = aa_v[s, pl.ds(k * 16, 16)] + pos_v[p, pl.ds(k * 16, 16)]
                    obuf[bidx, tb + l, pl.ds(k * 16, 16)] = \
                        (e - ml) * rl * gvec[k] + bvec[k]
            return 0

        lax.fori_loop(0, NCHUNK, chunk_body, 0)
        pltpu.make_async_copy(obuf.at[bidx, pl.ds(0, S)],
                              out_h.at[base_row + r], sem_out).start()
        return 0

    lax.fori_loop(0, RW, row_body, 0)
    pltpu.make_async_copy(obuf.at[0, pl.ds(0, S)],
                          out_h.at[base_row + RW - 2], sem_out).wait()
    pltpu.make_async_copy(obuf.at[1, pl.ds(0, S)],
                          out_h.at[base_row + RW - 1], sem_out).wait()


_emb = functools.partial(
    pl.kernel,
    out_type=jax.ShapeDtypeStruct((B, S, D), jnp.float32),
    mesh=plsc.VectorSubcoreMesh(core_axis_name="c", subcore_axis_name="s"),
    scratch_types=[
        pltpu.VMEM((VOCAB, D), jnp.float32),
        pltpu.VMEM((MAXPOS, D), jnp.float32),
        pltpu.VMEM((D,), jnp.float32),
        pltpu.VMEM((D,), jnp.float32),
        pltpu.VMEM((RW * S + 64,), jnp.int32),
        pltpu.VMEM((2 * SPAD,), jnp.int32),
        pltpu.VMEM((2 * SPAD,), jnp.float32),
        pltpu.VMEM((2 * SPAD,), jnp.float32),
        pltpu.VMEM((2, SPAD, D), jnp.float32),
        pltpu.SemaphoreType.DMA,
        pltpu.SemaphoreType.DMA,
        pltpu.SemaphoreType.DMA,
    ],
)(_body)


def _stats_body(aa_ref, pos_ref, m_ref, r_ref):
    aa = aa_ref[...]
    pos = pos_ref[...]
    cd = lax.dot_general(aa, pos, (((1,), (1,)), ((), ())),
                         preferred_element_type=jnp.float32)
    sa = jnp.sum(aa, axis=1, keepdims=True)
    qa = jnp.sum(aa * aa, axis=1, keepdims=True)
    sp = jnp.sum(pos, axis=1)
    qp = jnp.sum(pos * pos, axis=1)
    m = (sa + sp[None, :]) * (1.0 / D)
    q = (qa + qp[None, :] + 2.0 * cd) * (1.0 / D)
    v = jnp.maximum(q - m * m, 0.0) + EPS
    m_ref[...] = m
    r_ref[...] = lax.rsqrt(v)


_stats_tc = pl.pallas_call(
    _stats_body,
    out_shape=(jax.ShapeDtypeStruct((VOCAB, MAXPOS), jnp.float32),
               jax.ShapeDtypeStruct((VOCAB, MAXPOS), jnp.float32)),
)


def kernel(src, AA_emb, Pos_emb, ln_gamma, ln_beta):
    src_flat = src.reshape(-1).astype(jnp.int32)
    m_tab, r_tab = _stats_tc(AA_emb, Pos_emb)
    return _emb(src_flat, AA_emb, Pos_emb, ln_gamma, ln_beta,
                m_tab.reshape(-1), r_tab.reshape(-1))


# speedup vs baseline: 1.9868x; 1.1853x over previous
import functools

import jax
import jax.numpy as jnp
from jax import lax
from jax.experimental import pallas as pl
from jax.experimental.pallas import tpu as pltpu
from jax.experimental.pallas import tpu_sc as plsc

B, S, D = 1024, 200, 128
PAD = 21
VOCAB, MAXPOS = 32, 256
EPS = 1e-12
NC, NS = 2, 16
NW = NC * NS
RW = B // NW
SPAD = 208
NCHUNK = SPAD // 16
H0, H1 = 112, SPAD - 112


def _perm(x, idx):
    return x.at[idx].get(mode="promise_in_bounds")


def _prefix_sum(x):
    lane = lax.iota(jnp.int32, 16)
    for k in range(4):
        sh = _perm(x, jnp.maximum(lane - (1 << k), 0))
        x = x + jnp.where(lane >= (1 << k), sh, jnp.int32(0))
    return x


def _body(src_h, tab_h, out_h, src_v, pidx_v, obuf,
          sem_in, sem_g, sem_out):
    sid = lax.axis_index("s")
    wid = sid * NC + lax.axis_index("c")
    base_row = wid * RW

    cin = pltpu.make_async_copy(src_h.at[pl.ds(base_row * S, RW * S)],
                                src_v.at[pl.ds(0, RW * S)], sem_in)
    cin.start()
    cin.wait()

    last = jnp.full((16,), 15, jnp.int32)

    def compute_pair_idx(row):
        b = lax.rem(row, 2)

        def och(i, carry):
            tb = i * 16
            sv_raw = src_v[pl.ds(row * S + tb, 16)]
            sv = jnp.minimum(jnp.maximum(sv_raw, jnp.int32(0)),
                             jnp.int32(VOCAB - 1))
            m = jnp.where(sv_raw == PAD, jnp.int32(0), jnp.int32(1))
            c = _prefix_sum(m) + carry
            pv = jnp.where(m == 1, c, jnp.int32(0))
            pv = jnp.minimum(pv, jnp.int32(MAXPOS - 1))
            pidx_v[pl.ds(b * SPAD + tb, 16)] = sv * jnp.int32(MAXPOS) + pv
            return _perm(c, last)

        lax.fori_loop(0, NCHUNK, och, jnp.zeros((16,), jnp.int32))

    def gathers(row, slot):
        b = lax.rem(row, 2)
        return [
            pltpu.make_async_copy(
                tab_h.at[pidx_v.at[pl.ds(b * SPAD + off, ln)]],
                obuf.at[slot, pl.ds(off, ln)], sem_g)
            for off, ln in ((0, H0), (H0, H1))
        ]

    def out_copy(row, slot):
        return pltpu.make_async_copy(obuf.at[slot, pl.ds(0, S)],
                                     out_h.at[base_row + row], sem_out)

    compute_pair_idx(0)
    compute_pair_idx(1)
    for c in gathers(0, 0):
        c.start()
    for c in gathers(1, 1):
        c.start()

    def row_body(r, _):
        slot = lax.rem(r, 4)

        for c in gathers(r, slot):
            c.wait()
        out_copy(r, slot).start()

        @pl.when(r + 2 < RW)
        def _prefetch_next():
            nslot = lax.rem(r + 2, 4)

            @pl.when(r >= 2)
            def _wait_prev_out():
                out_copy(r - 2, nslot).wait()

            compute_pair_idx(r + 2)
            for c in gathers(r + 2, nslot):
                c.start()

        return 0

    lax.fori_loop(0, RW, row_body, 0)
    for rr in (RW - 4, RW - 3, RW - 2, RW - 1):
        out_copy(rr, lax.rem(rr, 4)).wait()


_emb = functools.partial(
    pl.kernel,
    out_type=jax.ShapeDtypeStruct((B, S, D), jnp.float32),
    mesh=plsc.VectorSubcoreMesh(core_axis_name="c", subcore_axis_name="s"),
    scratch_types=[
        pltpu.VMEM((RW * S + 64,), jnp.int32),
        pltpu.VMEM((2 * SPAD,), jnp.int32),
        pltpu.VMEM((4, SPAD, D), jnp.float32),
        pltpu.SemaphoreType.DMA,
        pltpu.SemaphoreType.DMA,
        pltpu.SemaphoreType.DMA,
    ],
)(_body)


def _table_body(aa_ref, pos_ref, ga_ref, be_ref, t_ref):
    aa = aa_ref[...]
    pos = pos_ref[...]
    g = ga_ref[...]
    bb = be_ref[...]
    e = aa[:, None, :] + pos[None, :, :]
    mean = jnp.mean(e, axis=-1, keepdims=True)
    var = jnp.mean(jnp.square(e - mean), axis=-1, keepdims=True)
    normed = (e - mean) * lax.rsqrt(var + EPS)
    t_ref[...] = normed * g[None, None, :] + bb[None, None, :]


_table_tc = pl.pallas_call(
    _table_body,
    out_shape=jax.ShapeDtypeStruct((VOCAB, MAXPOS, D), jnp.float32),
)


def kernel(src, AA_emb, Pos_emb, ln_gamma, ln_beta):
    src_flat = src.reshape(-1).astype(jnp.int32)
    tab = _table_tc(AA_emb, Pos_emb, ln_gamma, ln_beta)
    return _emb(src_flat, tab.reshape(VOCAB * MAXPOS, D))


# speedup vs baseline: 3.7006x; 1.8626x over previous
import functools

import jax
import jax.numpy as jnp
from jax import lax
from jax.experimental import pallas as pl
from jax.experimental.pallas import tpu as pltpu
from jax.experimental.pallas import tpu_sc as plsc

B, S, D = 1024, 200, 128
PAD = 21
VOCAB, MAXPOS = 32, 256
EPS = 1e-12
NC, NS = 2, 16
NW = NC * NS
RW = B // NW
HR = RW // 2
SPAD = 208
NCHUNK = SPAD // 16
NK = D // 16
H0, H1 = 112, S - 112


def _perm(x, idx):
    return x.at[idx].get(mode="promise_in_bounds")


def _splat_sum(x):
    lane = lax.iota(jnp.int32, 16)
    for k in range(4):
        x = x + _perm(x, lane ^ (1 << k))
    return x


def _prefix_sum(x):
    lane = lax.iota(jnp.int32, 16)
    for k in range(4):
        sh = _perm(x, jnp.maximum(lane - (1 << k), 0))
        x = x + jnp.where(lane >= (1 << k), sh, jnp.int32(0))
    return x


def _rsqrt(x):
    i = lax.bitcast_convert_type(x, jnp.int32)
    i = jnp.int32(0x5F3759DF) - lax.shift_right_logical(i, 1)
    y = lax.bitcast_convert_type(i, jnp.float32)
    for _ in range(3):
        y = y * (1.5 - 0.5 * x * y * y)
    return y


def _body(src_h, aa_h, pos_h, ga_h, be_h, tab_h, out_h,
          aa_v, pos_v, ga_v, be_v, src_v, pidx_v, obuf, dbuf,
          sem_in, sem_g, sem_ot, sem_od):
    wid = lax.axis_index("s") * NC + lax.axis_index("c")
    base_row = wid * RW

    stage = [
        pltpu.make_async_copy(aa_h, aa_v, sem_in),
        pltpu.make_async_copy(pos_h, pos_v, sem_in),
        pltpu.make_async_copy(ga_h, ga_v, sem_in),
        pltpu.make_async_copy(be_h, be_v, sem_in),
        pltpu.make_async_copy(src_h.at[pl.ds(base_row * S, RW * S)],
                              src_v.at[pl.ds(0, RW * S)], sem_in),
    ]
    for c in stage:
        c.start()
    for c in stage:
        c.wait()

    gvec = [ga_v[pl.ds(k * 16, 16)] for k in range(NK)]
    bvec = [be_v[pl.ds(k * 16, 16)] for k in range(NK)]
    last = jnp.full((16,), 15, jnp.int32)

    def pos_ids(row, i, carry):
        tb = i * 16
        sv_raw = src_v[pl.ds(row * S + tb, 16)]
        sv = jnp.minimum(jnp.maximum(sv_raw, jnp.int32(0)),
                         jnp.int32(VOCAB - 1))
        m = jnp.where(sv_raw == PAD, jnp.int32(0), jnp.int32(1))
        c = _prefix_sum(m) + carry
        pv = jnp.where(m == 1, c, jnp.int32(0))
        pv = jnp.minimum(pv, jnp.int32(MAXPOS - 1))
        return sv, pv, c

    def compute_pair_idx(d):
        def och(i, carry):
            sv, pv, c = pos_ids(HR + d, i, carry)
            pidx_v[pl.ds(d * SPAD + i * 16, 16)] = \
                sv * jnp.int32(MAXPOS) + pv
            return _perm(c, last)

        lax.fori_loop(0, NCHUNK, och, jnp.zeros((16,), jnp.int32))

    def gathers(d):
        return [
            pltpu.make_async_copy(
                tab_h.at[pidx_v.at[pl.ds(d * SPAD + off, ln)]],
                dbuf.at[pl.ds(off, ln)], sem_g)
            for off, ln in ((0, H0), (H0, H1))
        ]

    def out_dma(row, buf_ref, sem):
        return pltpu.make_async_copy(buf_ref, out_h.at[base_row + row], sem)

    compute_pair_idx(0)
    for c in gathers(0):
        c.start()

    def row_body(r, _):
        bidx = lax.rem(r, 2)

        @pl.when(r + 1 < HR)
        def _next_idx():
            compute_pair_idx(r + 1)

        @pl.when(r >= 2)
        def _wait_prev():
            out_dma(r - 2, obuf.at[bidx, pl.ds(0, S)], sem_ot).wait()

        def chunk_body(i, carry):
            tb = i * 16
            sv, pv, c = pos_ids(r, i, carry)
            for l in range(16):
                s = sv[l]
                p = pv[l]
                e = [aa_v[s, pl.ds(k * 16, 16)] + pos_v[p, pl.ds(k * 16, 16)]
                     for k in range(NK)]
                tot = ((e[0] + e[1]) + (e[2] + e[3])) + \
                      ((e[4] + e[5]) + (e[6] + e[7]))
                sq = [ek * ek for ek in e]
                sqt = ((sq[0] + sq[1]) + (sq[2] + sq[3])) + \
                      ((sq[4] + sq[5]) + (sq[6] + sq[7]))
                mean = _splat_sum(tot) * (1.0 / D)
                ex2 = _splat_sum(sqt) * (1.0 / D)
                var = jnp.maximum(ex2 - mean * mean, 0.0) + EPS
                rstd = _rsqrt(var)
                for k in range(NK):
                    obuf[bidx, tb + l, pl.ds(k * 16, 16)] = \
                        (e[k] - mean) * rstd * gvec[k] + bvec[k]
            return _perm(c, last)

        lax.fori_loop(0, NCHUNK, chunk_body, jnp.zeros((16,), jnp.int32))
        out_dma(r, obuf.at[bidx, pl.ds(0, S)], sem_ot).start()

        for c in gathers(r):
            c.wait()
        out_dma(HR + r, dbuf, sem_od).start()

        @pl.when(r + 1 < HR)
        def _next_gather():
            out_dma(HR + r, dbuf, sem_od).wait()
            for c in gathers(r + 1):
                c.start()

        return 0

    lax.fori_loop(0, HR, row_body, 0)
    out_dma(HR - 2, obuf.at[0, pl.ds(0, S)], sem_ot).wait()
    out_dma(HR - 1, obuf.at[1, pl.ds(0, S)], sem_ot).wait()
    out_dma(RW - 1, dbuf, sem_od).wait()


_emb = functools.partial(
    pl.kernel,
    out_type=jax.ShapeDtypeStruct((B, S, D), jnp.float32),
    mesh=plsc.VectorSubcoreMesh(core_axis_name="c", subcore_axis_name="s"),
    scratch_types=[
        pltpu.VMEM((VOCAB, D), jnp.float32),
        pltpu.VMEM((MAXPOS, D), jnp.float32),
        pltpu.VMEM((D,), jnp.float32),
        pltpu.VMEM((D,), jnp.float32),
        pltpu.VMEM((RW * S + 64,), jnp.int32),
        pltpu.VMEM((HR * SPAD,), jnp.int32),
        pltpu.VMEM((2, SPAD, D), jnp.float32),
        pltpu.VMEM((S, D), jnp.float32),
        pltpu.SemaphoreType.DMA,
        pltpu.SemaphoreType.DMA,
        pltpu.SemaphoreType.DMA,
        pltpu.SemaphoreType.DMA,
    ],
)(_body)


def _table_body(aa_ref, pos_ref, ga_ref, be_ref, t_ref):
    aa = aa_ref[...]
    pos = pos_ref[...]
    g = ga_ref[...]
    bb = be_ref[...]
    e = aa[:, None, :] + pos[None, :, :]
    mean = jnp.mean(e, axis=-1, keepdims=True)
    var = jnp.mean(jnp.square(e - mean), axis=-1, keepdims=True)
    normed = (e - mean) * lax.rsqrt(var + EPS)
    t_ref[...] = normed * g[None, None, :] + bb[None, None, :]


_table_tc = pl.pallas_call(
    _table_body,
    out_shape=jax.ShapeDtypeStruct((VOCAB, MAXPOS, D), jnp.float32),
)


def kernel(src, AA_emb, Pos_emb, ln_gamma, ln_beta):
    src_flat = src.reshape(-1).astype(jnp.int32)
    tab = _table_tc(AA_emb, Pos_emb, ln_gamma, ln_beta)
    return _emb(src_flat, AA_emb, Pos_emb, ln_gamma, ln_beta,
                tab.reshape(VOCAB * MAXPOS, D))


# speedup vs baseline: 3.9293x; 1.0618x over previous
import functools

import jax
import jax.numpy as jnp
from jax import lax
from jax.experimental import pallas as pl
from jax.experimental.pallas import tpu as pltpu
from jax.experimental.pallas import tpu_sc as plsc

B, S, D = 1024, 200, 128
PAD = 21
VOCAB, MAXPOS = 32, 256
EPS = 1e-12
NC, NS = 2, 16
NW = NC * NS
RW = B // NW
HR = RW // 2
SPAD = 208
NCHUNK = SPAD // 16
NK = D // 16
H0, H1 = 112, S - 112


def _perm(x, idx):
    return x.at[idx].get(mode="promise_in_bounds")


def _splat_sum(x):
    lane = lax.iota(jnp.int32, 16)
    for k in range(4):
        x = x + _perm(x, lane ^ (1 << k))
    return x


def _prefix_sum(x):
    lane = lax.iota(jnp.int32, 16)
    for k in range(4):
        sh = _perm(x, jnp.maximum(lane - (1 << k), 0))
        x = x + jnp.where(lane >= (1 << k), sh, jnp.int32(0))
    return x


def _rsqrt(x):
    i = lax.bitcast_convert_type(x, jnp.int32)
    i = jnp.int32(0x5F3759DF) - lax.shift_right_logical(i, 1)
    y = lax.bitcast_convert_type(i, jnp.float32)
    for _ in range(2):
        y = y * (1.5 - 0.5 * x * y * y)
    return y


def _body(src_h, aa_h, pos_h, ga_h, be_h, tab_h, out_h,
          aa_v, pos_v, ga_v, be_v, src_v, pidx_v, obuf, dbuf,
          sem_in, sem_g, sem_ot, sem_od):
    wid = lax.axis_index("s") * NC + lax.axis_index("c")
    base_row = wid * RW

    stage = [
        pltpu.make_async_copy(aa_h, aa_v, sem_in),
        pltpu.make_async_copy(pos_h, pos_v, sem_in),
        pltpu.make_async_copy(ga_h, ga_v, sem_in),
        pltpu.make_async_copy(be_h, be_v, sem_in),
        pltpu.make_async_copy(src_h.at[pl.ds(base_row * S, RW * S)],
                              src_v.at[pl.ds(0, RW * S)], sem_in),
    ]
    for c in stage:
        c.start()
    for c in stage:
        c.wait()

    gvec = [ga_v[pl.ds(k * 16, 16)] for k in range(NK)]
    bvec = [be_v[pl.ds(k * 16, 16)] for k in range(NK)]
    last = jnp.full((16,), 15, jnp.int32)

    def pos_ids(row, i, carry):
        tb = i * 16
        sv_raw = src_v[pl.ds(row * S + tb, 16)]
        sv = jnp.minimum(jnp.maximum(sv_raw, jnp.int32(0)),
                         jnp.int32(VOCAB - 1))
        m = jnp.where(sv_raw == PAD, jnp.int32(0), jnp.int32(1))
        c = _prefix_sum(m) + carry
        pv = jnp.where(m == 1, c, jnp.int32(0))
        pv = jnp.minimum(pv, jnp.int32(MAXPOS - 1))
        return sv, pv, c

    def compute_pair_idx(d):
        def och(i, carry):
            sv, pv, c = pos_ids(HR + d, i, carry)
            pidx_v[pl.ds(d * SPAD + i * 16, 16)] = \
                sv * jnp.int32(MAXPOS) + pv
            return _perm(c, last)

        lax.fori_loop(0, NCHUNK, och, jnp.zeros((16,), jnp.int32))

    def gathers(d):
        return [
            pltpu.make_async_copy(
                tab_h.at[pidx_v.at[pl.ds(d * SPAD + off, ln)]],
                dbuf.at[pl.ds(off, ln)], sem_g)
            for off, ln in ((0, H0), (H0, H1))
        ]

    def out_dma(row, buf_ref, sem):
        return pltpu.make_async_copy(buf_ref, out_h.at[base_row + row], sem)

    compute_pair_idx(0)
    for c in gathers(0):
        c.start()

    def row_body(r, _):
        bidx = lax.rem(r, 2)

        @pl.when(r + 1 < HR)
        def _next_idx():
            compute_pair_idx(r + 1)

        @pl.when(r >= 2)
        def _wait_prev():
            out_dma(r - 2, obuf.at[bidx, pl.ds(0, S)], sem_ot).wait()

        def chunk_body(i, carry):
            tb = i * 16
            sv, pv, c = pos_ids(r, i, carry)
            for l in range(16):
                s = sv[l]
                p = pv[l]
                e = [aa_v[s, pl.ds(k * 16, 16)] + pos_v[p, pl.ds(k * 16, 16)]
                     for k in range(NK)]
                tot = ((e[0] + e[1]) + (e[2] + e[3])) + \
                      ((e[4] + e[5]) + (e[6] + e[7]))
                sq = [ek * ek for ek in e]
                sqt = ((sq[0] + sq[1]) + (sq[2] + sq[3])) + \
                      ((sq[4] + sq[5]) + (sq[6] + sq[7]))
                mean = _splat_sum(tot) * (1.0 / D)
                ex2 = _splat_sum(sqt) * (1.0 / D)
                var = jnp.maximum(ex2 - mean * mean, 0.0) + EPS
                rstd = _rsqrt(var)
                for k in range(NK):
                    obuf[bidx, tb + l, pl.ds(k * 16, 16)] = \
                        (e[k] - mean) * rstd * gvec[k] + bvec[k]
            return _perm(c, last)

        lax.fori_loop(0, NCHUNK, chunk_body, jnp.zeros((16,), jnp.int32))
        out_dma(r, obuf.at[bidx, pl.ds(0, S)], sem_ot).start()

        for c in gathers(r):
            c.wait()
        out_dma(HR + r, dbuf, sem_od).start()

        @pl.when(r + 1 < HR)
        def _next_gather():
            out_dma(HR + r, dbuf, sem_od).wait()
            for c in gathers(r + 1):
                c.start()

        return 0

    lax.fori_loop(0, HR, row_body, 0)
    out_dma(HR - 2, obuf.at[0, pl.ds(0, S)], sem_ot).wait()
    out_dma(HR - 1, obuf.at[1, pl.ds(0, S)], sem_ot).wait()
    out_dma(RW - 1, dbuf, sem_od).wait()


_emb = functools.partial(
    pl.kernel,
    out_type=jax.ShapeDtypeStruct((B, S, D), jnp.float32),
    mesh=plsc.VectorSubcoreMesh(core_axis_name="c", subcore_axis_name="s"),
    scratch_types=[
        pltpu.VMEM((VOCAB, D), jnp.float32),
        pltpu.VMEM((MAXPOS, D), jnp.float32),
        pltpu.VMEM((D,), jnp.float32),
        pltpu.VMEM((D,), jnp.float32),
        pltpu.VMEM((RW * S + 64,), jnp.int32),
        pltpu.VMEM((HR * SPAD,), jnp.int32),
        pltpu.VMEM((2, SPAD, D), jnp.float32),
        pltpu.VMEM((S, D), jnp.float32),
        pltpu.SemaphoreType.DMA,
        pltpu.SemaphoreType.DMA,
        pltpu.SemaphoreType.DMA,
        pltpu.SemaphoreType.DMA,
    ],
)(_body)


def _table_body(aa_ref, pos_ref, ga_ref, be_ref, t_ref):
    aa = aa_ref[...]
    pos = pos_ref[...]
    g = ga_ref[...]
    bb = be_ref[...]
    e = aa[:, None, :] + pos[None, :, :]
    mean = jnp.mean(e, axis=-1, keepdims=True)
    var = jnp.mean(jnp.square(e - mean), axis=-1, keepdims=True)
    normed = (e - mean) * lax.rsqrt(var + EPS)
    t_ref[...] = normed * g[None, None, :] + bb[None, None, :]


_table_tc = pl.pallas_call(
    _table_body,
    out_shape=jax.ShapeDtypeStruct((VOCAB, MAXPOS, D), jnp.float32),
)


def kernel(src, AA_emb, Pos_emb, ln_gamma, ln_beta):
    src_flat = src.reshape(-1).astype(jnp.int32)
    tab = _table_tc(AA_emb, Pos_emb, ln_gamma, ln_beta)
    return _emb(src_flat, AA_emb, Pos_emb, ln_gamma, ln_beta,
                tab.reshape(VOCAB * MAXPOS, D))
